# baseline (speedup 1.0000x reference)
import functools

import numpy as np
import jax
import jax.numpy as jnp
from jax import lax
from jax.experimental import pallas as pl
from jax.experimental.pallas import tpu as pltpu
from jax.experimental.pallas import tpu_sc as plsc

_FIELD_DIMS = [40000] * 26
_NF = 26
_EMB = 32
_BATCH = 16384
_ROWS = _BATCH * _NF
_NW = 32
_BPW = _ROWS // _NW
_BATW = _BATCH // _NW
_CHUNK = 128
_KB = _BATW // _CHUNK
_NCH = _BPW // _CHUNK
_LANES = 16
_VPC = _CHUNK // _LANES

_FDIM = _FIELD_DIMS[0]
_TOTAL = _NF * _FDIM
_NB = 8
_NR = _NCH // _NB

_mesh = plsc.VectorSubcoreMesh(core_axis_name="c", subcore_axis_name="s")

_QUART = _TOTAL // 4
_PANEL = 3200


def _tc_repack_body(in_ref, o_ref):
    a = in_ref[...]
    for j in range(4):
        o_ref[:, j * 32:(j + 1) * 32] = a[:, j, :].T


_tc_repack = pl.pallas_call(
    _tc_repack_body,
    grid=((_QUART + _PANEL - 1) // _PANEL,),
    in_specs=[pl.BlockSpec((_EMB, 4, _PANEL), lambda c: (0, 0, c))],
    out_specs=pl.BlockSpec((_PANEL, 128), lambda c: (c, 0)),
    out_shape=jax.ShapeDtypeStruct((_QUART, 128), jnp.float32),
)


@functools.partial(
    pl.kernel,
    mesh=_mesh,
    out_type=jax.ShapeDtypeStruct((_ROWS, _EMB), jnp.float32),
    compiler_params=pltpu.CompilerParams(use_tc_tiling_on_sc=False),
    scratch_types=[
        pltpu.VMEM((_NF * _BATW,), jnp.int32),
        pltpu.VMEM((_NCH, _CHUNK), jnp.int32),
        pltpu.VMEM((_NCH, _CHUNK), jnp.int32),
        pltpu.VMEM((_NB, _CHUNK, _EMB), jnp.float32),
        pltpu.SemaphoreType.DMA,
        [pltpu.SemaphoreType.DMA] * _NB,
        [pltpu.SemaphoreType.DMA] * _NB,
    ],
)
def _emb_gather(xt_hbm, w_hbm, out_hbm, xs_v, gidx_v, oidx_v, bufs,
                xsem, gsems, ssems):
    cid = lax.axis_index("c")
    sid = lax.axis_index("s")
    wid = sid * 2 + cid
    b0 = wid * _BATW

    stages = [
        pltpu.async_copy(
            xt_hbm.at[f, pl.ds(b0, _BATW)],
            xs_v.at[pl.ds(f * _BATW, _BATW)], xsem)
        for f in range(_NF)
    ]

    ramp = lax.iota(jnp.int32, _LANES) * _NF
    for h in stages:
        h.wait()

    def _fix(j, _):
        f = lax.div(j, _KB)
        k = lax.rem(j, _KB)
        off = f * _FDIM
        o0 = (b0 + k * _CHUNK) * _NF + f
        for t in range(_VPC):
            s = pl.ds(t * _LANES, _LANES)
            v = xs_v[pl.ds(f * _BATW + k * _CHUNK + t * _LANES,
                           _LANES)] + off
            d = (jnp.where(v >= _QUART, 1, 0)
                 + jnp.where(v >= 2 * _QUART, 1, 0)
                 + jnp.where(v >= 3 * _QUART, 1, 0))
            gidx_v[j, s] = v * 4 - d * (4 * _QUART - 1)
            oidx_v[j, s] = ramp + (o0 + t * _LANES * _NF)
        return ()

    lax.fori_loop(0, _NCH, _fix, ())

    def _round(r, _):
        handles = []
        for t in range(_NB):
            @pl.when(r > 0)
            def _drain(t=t):
                pltpu.make_async_copy(
                    bufs.at[t], out_hbm.at[oidx_v.at[0]], ssems[t]
                ).wait()
            j = r * _NB + t
            handles.append(
                pltpu.async_copy(w_hbm.at[gidx_v.at[j]], bufs.at[t], gsems[t]))
        for t in range(_NB):
            handles[t].wait()
            j = r * _NB + t
            pltpu.async_copy(
                bufs.at[t], out_hbm.at[oidx_v.at[j]], ssems[t])
        return ()

    lax.fori_loop(0, _NR, _round, ())

    for t in range(_NB):
        pltpu.make_async_copy(
            bufs.at[t], out_hbm.at[oidx_v.at[0]], ssems[t]).wait()


def kernel(x, W):
    xt = x.astype(jnp.int32).T
    wt = W.astype(jnp.float32).T
    w_rm = _tc_repack(wt.reshape(_EMB, 4, _QUART)).reshape(_TOTAL, _EMB)
    out = _emb_gather(xt, w_rm)
    return out.reshape(_BATCH, _NF, _EMB)


# speedup vs baseline: 1.2196x; 1.2196x over previous
import functools

import numpy as np
import jax
import jax.numpy as jnp
from jax import lax
from jax.experimental import pallas as pl
from jax.experimental.pallas import tpu as pltpu
from jax.experimental.pallas import tpu_sc as plsc

_FIELD_DIMS = [40000] * 26
_NF = 26
_EMB = 32
_BATCH = 16384
_ROWS = _BATCH * _NF
_NW = 32
_BPW = _ROWS // _NW
_BATW = _BATCH // _NW
_CHUNK = 128
_KB = _BATW // _CHUNK
_NCH = _BPW // _CHUNK
_LANES = 16
_VPC = _CHUNK // _LANES

_FDIM = _FIELD_DIMS[0]
_NB = 8
_NR = _NCH // _NB

_mesh = plsc.VectorSubcoreMesh(core_axis_name="c", subcore_axis_name="s")


@functools.partial(
    pl.kernel,
    mesh=_mesh,
    out_type=jax.ShapeDtypeStruct((_ROWS, _EMB), jnp.float32),
    compiler_params=pltpu.CompilerParams(use_tc_tiling_on_sc=False),
    scratch_types=[
        pltpu.VMEM((_NF * _BATW,), jnp.int32),
        pltpu.VMEM((_NCH, _CHUNK), jnp.int32),
        pltpu.VMEM((_NCH, _CHUNK), jnp.int32),
        pltpu.VMEM((_NB, _CHUNK, _EMB), jnp.float32),
        pltpu.SemaphoreType.DMA,
        [pltpu.SemaphoreType.DMA] * _NB,
        [pltpu.SemaphoreType.DMA] * _NB,
    ],
)
def _emb_gather(xt_hbm, w_hbm, out_hbm, xs_v, gidx_v, oidx_v, bufs,
                xsem, gsems, ssems):
    cid = lax.axis_index("c")
    sid = lax.axis_index("s")
    wid = sid * 2 + cid
    b0 = wid * _BATW

    stages = [
        pltpu.async_copy(
            xt_hbm.at[f, pl.ds(b0, _BATW)],
            xs_v.at[pl.ds(f * _BATW, _BATW)], xsem)
        for f in range(_NF)
    ]

    ramp = lax.iota(jnp.int32, _LANES) * _NF
    for h in stages:
        h.wait()

    def _fix(j, _):
        f = lax.div(j, _KB)
        k = lax.rem(j, _KB)
        off = f * _FDIM
        o0 = (b0 + k * _CHUNK) * _NF + f
        for t in range(_VPC):
            s = pl.ds(t * _LANES, _LANES)
            gidx_v[j, s] = xs_v[pl.ds(f * _BATW + k * _CHUNK + t * _LANES,
                                      _LANES)] + off
            oidx_v[j, s] = ramp + (o0 + t * _LANES * _NF)
        return ()

    lax.fori_loop(0, _NCH, _fix, ())

    def _round(r, _):
        handles = []
        for t in range(_NB):
            @pl.when(r > 0)
            def _drain(t=t):
                pltpu.make_async_copy(
                    bufs.at[t], out_hbm.at[oidx_v.at[0]], ssems[t]
                ).wait()
            j = r * _NB + t
            handles.append(
                pltpu.async_copy(w_hbm.at[gidx_v.at[j]], bufs.at[t], gsems[t]))
        for t in range(_NB):
            handles[t].wait()
            j = r * _NB + t
            pltpu.async_copy(
                bufs.at[t], out_hbm.at[oidx_v.at[j]], ssems[t])
        return ()

    lax.fori_loop(0, _NR, _round, ())

    for t in range(_NB):
        pltpu.make_async_copy(
            bufs.at[t], out_hbm.at[oidx_v.at[0]], ssems[t]).wait()


def kernel(x, W):
    xt = x.astype(jnp.int32).T
    out = _emb_gather(xt, W.astype(jnp.float32))
    return out.reshape(_BATCH, _NF, _EMB)
